# baseline (speedup 1.0000x reference)
import functools

import jax
import jax.numpy as jnp
from jax import lax
from jax.experimental import pallas as pl
from jax.experimental.pallas import tpu as pltpu
from jax.experimental.pallas import tpu_sc as plsc

_NC, _NS = 2, 16
_NW = _NC * _NS
_L = 16
_TR = 8
_BL = 128
_HB = 64


@functools.partial(jax.jit, static_argnums=(2,))
def _embedding_lookup(tpad, idxr, dims):
    bsz, seq, d, dp = dims
    nti = seq // _TR
    nblk = nti * 2
    mesh = plsc.VectorSubcoreMesh(core_axis_name="c", subcore_axis_name="s")

    @functools.partial(
        pl.kernel,
        out_type=jax.ShapeDtypeStruct((d, nti, _NW, _TR, _BL), tpad.dtype),
        mesh=mesh,
        compiler_params=pltpu.CompilerParams(
            use_tc_tiling_on_sc=False, needs_layout_passes=False),
        scratch_types=[
            pltpu.VMEM((_HB * _TR,), jnp.int32),
            pltpu.VMEM((_HB * _TR,), jnp.int32),
            pltpu.VMEM((_HB * _TR, dp), jnp.float32),
            pltpu.VMEM((_HB * _TR, dp), jnp.float32),
            pltpu.VMEM((d, _TR, _HB), jnp.float32),
            pltpu.VMEM((d, _TR, _HB), jnp.float32),
            pltpu.SemaphoreType.DMA,
            pltpu.SemaphoreType.DMA,
            pltpu.SemaphoreType.DMA,
            pltpu.SemaphoreType.DMA,
        ],
    )
    def body(table_hbm, idx_hbm, out_hbm,
             ix0, ix1, gb0, gb1, tb0, tb1, g0, g1, w0, w1):
        wid = lax.axis_index("s") * _NC + lax.axis_index("c")
        ixs = (ix0, ix1)
        gbs = (gb0, gb1)
        tbs = (tb0, tb1)
        gsems = (g0, g1)
        wsems = (w0, w1)

        lane_rows = lax.iota(jnp.int32, _L) * _TR

        def load_and_fire(m, par):
            pltpu.sync_copy(idx_hbm.at[wid * nblk + m], ixs[par])
            pltpu.async_copy(table_hbm.at[ixs[par]], gbs[par], gsems[par])

        def drain_writes(m, par):
            ti = m // 2
            half = m % 2

            @pl.loop(0, d)
            def _(e):
                pltpu.make_async_copy(
                    tbs[par].at[e],
                    out_hbm.at[e, ti, wid, slice(None),
                               pl.ds(half * _HB, _HB)],
                    wsems[par]).wait()

        def step(m, par):
            ti = m // 2
            half = m % 2
            @pl.when(m >= 2)
            def _():
                drain_writes(m - 2, par)

            pltpu.make_async_copy(
                table_hbm.at[ixs[par]], gbs[par], gsems[par]).wait()

            @plsc.parallel_loop(0, d, 1, unroll=5)
            def _(e):
                cols = jnp.full((_L,), e, jnp.int32)
                for tr in range(_TR):
                    for blg in range(_HB // _L):
                        rows = lane_rows + (blg * (_L * _TR) + tr)
                        vals = plsc.load_gather(gbs[par], [rows, cols])
                        tbs[par][e, tr, pl.ds(blg * _L, _L)] = vals
                pltpu.async_copy(
                    tbs[par].at[e],
                    out_hbm.at[e, ti, wid, slice(None),
                               pl.ds(half * _HB, _HB)],
                    wsems[par])

            @pl.when(m + 2 < nblk)
            def _():
                load_and_fire(m + 2, par)

        for par in range(2):
            load_and_fire(par, par)

        @pl.loop(0, nblk, step=2)
        def _(m):
            step(m, 0)
            step(m + 1, 1)

        for par in range(2):
            drain_writes(nblk - 2 + par, par)

    return body(tpad, idxr)


def kernel(table, input):
    bsz, seq = input.shape
    v, d = table.shape
    dp = (d + 7) // 8 * 8
    tpad = jnp.pad(table, ((0, 0), (0, dp - d)))
    idxr = (input.astype(jnp.int32)
            .reshape(_NW, 2, _HB, seq // _TR, _TR)
            .transpose(0, 3, 1, 2, 4)
            .reshape(_NW * (seq // _TR) * 2, _HB * _TR))
    out5 = _embedding_lookup(tpad, idxr, (bsz, seq, d, dp))
    return jnp.transpose(out5, (2, 4, 1, 3, 0)).reshape(bsz, seq, d)


# speedup vs baseline: 1.7382x; 1.7382x over previous
import functools

import jax
import jax.numpy as jnp
from jax import lax
from jax.experimental import pallas as pl
from jax.experimental.pallas import tpu as pltpu
from jax.experimental.pallas import tpu_sc as plsc

_NC, _NS = 2, 16
_NW = _NC * _NS
_CHUNK = 128
_WIDE = 128
_NBUF = 4


@functools.partial(jax.jit, static_argnums=(2, 3, 4))
def _embedding_lookup(tpad, idx2d, n, dp, nch):
    per_w = n // _NW
    mesh = plsc.VectorSubcoreMesh(core_axis_name="c", subcore_axis_name="s")

    @functools.partial(
        pl.kernel,
        out_type=jax.ShapeDtypeStruct((n, _WIDE), tpad.dtype),
        mesh=mesh,
        compiler_params=pltpu.CompilerParams(use_tc_tiling_on_sc=False),
        scratch_types=[
            pltpu.VMEM((nch, _CHUNK), jnp.int32),
        ] + [pltpu.VMEM((_CHUNK, dp), jnp.float32)] * _NBUF
          + [pltpu.SemaphoreType.DMA] * (2 * _NBUF),
    )
    def body(table_hbm, idx_hbm, out_hbm, idx_v, *rest):
        bufs = rest[:_NBUF]
        gsems = rest[_NBUF:2 * _NBUF]
        osems = rest[2 * _NBUF:3 * _NBUF]
        wid = lax.axis_index("s") * _NC + lax.axis_index("c")
        base = wid * per_w

        def out_slab(j):
            return out_hbm.at[pl.ds(base + j * _CHUNK, _CHUNK), pl.ds(0, dp)]

        pltpu.sync_copy(idx_hbm.at[pl.ds(wid * nch, nch)], idx_v)

        for b in range(_NBUF):
            pltpu.async_copy(table_hbm.at[idx_v.at[b]], bufs[b], gsems[b])

        steps = nch // _NBUF

        @pl.loop(0, steps)
        def _(g):
            j0 = g * _NBUF
            for b in range(_NBUF):
                pltpu.make_async_copy(
                    table_hbm.at[idx_v.at[j0 + b]], bufs[b], gsems[b]).wait()
                pltpu.async_copy(bufs[b], out_slab(j0 + b), osems[b])
            for b in range(_NBUF):

                @pl.when(g < steps - 1)
                def _():
                    pltpu.make_async_copy(
                        bufs[b], out_slab(j0 + b), osems[b]).wait()
                    pltpu.async_copy(
                        table_hbm.at[idx_v.at[j0 + b + _NBUF]],
                        bufs[b], gsems[b])

        for b in range(_NBUF):
            pltpu.make_async_copy(
                bufs[b], out_slab(nch - _NBUF + b), osems[b]).wait()

    return body(tpad, idx2d)


def kernel(table, input):
    bsz, seq = input.shape
    v, d = table.shape
    dp = (d + 7) // 8 * 8
    n = bsz * seq
    tpad = jnp.pad(table, ((0, 0), (0, dp - d)))
    idx2d = input.reshape(n // _CHUNK, _CHUNK).astype(jnp.int32)
    nch = (n // _NW) // _CHUNK
    out = _embedding_lookup(tpad, idx2d, n, dp, nch)
    return out[:, :d].reshape(bsz, seq, d)


# speedup vs baseline: 1.7463x; 1.0046x over previous
import functools

import jax
import jax.numpy as jnp
from jax import lax
from jax.experimental import pallas as pl
from jax.experimental.pallas import tpu as pltpu
from jax.experimental.pallas import tpu_sc as plsc

_NC, _NS = 2, 16
_NW = _NC * _NS
_CHUNK = 128
_WIDE = 128
_NBUF = 8


@functools.partial(jax.jit, static_argnums=(2, 3, 4))
def _embedding_lookup(tpad, idx2d, n, dp, nch):
    per_w = n // _NW
    mesh = plsc.VectorSubcoreMesh(core_axis_name="c", subcore_axis_name="s")

    @functools.partial(
        pl.kernel,
        out_type=jax.ShapeDtypeStruct((n, _WIDE), tpad.dtype),
        mesh=mesh,
        compiler_params=pltpu.CompilerParams(use_tc_tiling_on_sc=False),
        scratch_types=[
            pltpu.VMEM((nch, _CHUNK), jnp.int32),
        ] + [pltpu.VMEM((_CHUNK, dp), jnp.float32)] * _NBUF
          + [pltpu.SemaphoreType.DMA] * (2 * _NBUF),
    )
    def body(table_hbm, idx_hbm, out_hbm, idx_v, *rest):
        bufs = rest[:_NBUF]
        gsems = rest[_NBUF:2 * _NBUF]
        osems = rest[2 * _NBUF:3 * _NBUF]
        wid = lax.axis_index("s") * _NC + lax.axis_index("c")
        base = wid * per_w

        def out_slab(j):
            return out_hbm.at[pl.ds(base + j * _CHUNK, _CHUNK), pl.ds(0, dp)]

        pltpu.sync_copy(idx_hbm.at[pl.ds(wid * nch, nch)], idx_v)

        for b in range(_NBUF):
            pltpu.async_copy(table_hbm.at[idx_v.at[b]], bufs[b], gsems[b])

        steps = nch // _NBUF

        @pl.loop(0, steps)
        def _(g):
            j0 = g * _NBUF
            for b in range(_NBUF):
                pltpu.make_async_copy(
                    table_hbm.at[idx_v.at[j0 + b]], bufs[b], gsems[b]).wait()
                pltpu.async_copy(bufs[b], out_slab(j0 + b), osems[b])
            for b in range(_NBUF):

                @pl.when(g < steps - 1)
                def _():
                    pltpu.make_async_copy(
                        bufs[b], out_slab(j0 + b), osems[b]).wait()
                    pltpu.async_copy(
                        table_hbm.at[idx_v.at[j0 + b + _NBUF]],
                        bufs[b], gsems[b])

        for b in range(_NBUF):
            pltpu.make_async_copy(
                bufs[b], out_slab(nch - _NBUF + b), osems[b]).wait()

    return body(tpad, idx2d)


def kernel(table, input):
    bsz, seq = input.shape
    v, d = table.shape
    dp = (d + 7) // 8 * 8
    n = bsz * seq
    tpad = jnp.pad(table, ((0, 0), (0, dp - d)))
    idx2d = input.reshape(n // _CHUNK, _CHUNK).astype(jnp.int32)
    nch = (n // _NW) // _CHUNK
    out = _embedding_lookup(tpad, idx2d, n, dp, nch)
    return out[:, :d].reshape(bsz, seq, d)
